# baseline (speedup 1.0000x reference)
import functools
import jax
import jax.numpy as jnp
from jax import lax
from jax.experimental import pallas as pl
from jax.experimental.pallas import tpu as pltpu
from jax.experimental.pallas import tpu_sc as plsc

N_NODES = 10000
N_EDGES = 320000
D_FEAT = 128
D_EDGE = 16
D_OUT = 128

NC = 2
NS = 16
HALF = N_NODES // NC
EPT = N_EDGES // NS
CH = 80
NCHUNK = EPT // CH
ACC_ROWS = 10240
TRASH = N_NODES
ZR = 32


def _p_body(x_ref, w_ref, p_ref):
    p_ref[...] = jnp.dot(x_ref[...], w_ref[0], preferred_element_type=jnp.float32)


def _a_body(eaT_ref, w_ref, b_ref, a_ref):
    a_ref[...] = (
        jax.lax.dot_general(eaT_ref[...], w_ref[0],
                            dimension_numbers=(((0,), (0,)), ((), ())),
                            preferred_element_type=jnp.float32)
        + b_ref[0]
    )


def _sc_body(p_hbm, a_hbm, ei_hbm, out_hbm,
             rv0, cv0, rv1, cv1,
             gidx0, aidx0, sidx0, gidx1, aidx1, sidx1,
             pbuf0, abuf0, pbuf1, abuf1,
             zbuf, acc, sem0, sem1, rcs0, rcs1):
    c = lax.axis_index("c")
    s = lax.axis_index("s")
    base = c * HALF
    iota16 = lax.iota(jnp.int32, 16)

    def _zrow(i, _):
        for j in range(8):
            zbuf[i, pl.ds(j * 16, 16)] = jnp.zeros((16,), jnp.float32)
        return 0
    lax.fori_loop(0, ZR, _zrow, 0)
    rows_per_tile = ACC_ROWS // NS
    z0 = s * rows_per_tile
    for t in range(rows_per_tile // ZR):
        pltpu.sync_copy(zbuf, acc.at[pl.ds(z0 + t * ZR, ZR)])
    plsc.subcore_barrier()

    def _rc(i, rv, cv, rcs):
        e0 = s * EPT + i * CH
        pltpu.async_copy(ei_hbm.at[pl.ds(e0, CH)], rv, rcs)
        pltpu.async_copy(ei_hbm.at[pl.ds(N_EDGES + e0, CH)], cv, rcs)

    def _gissue(i, rv, cv, rcs, gidx, aidx, sidx, pbuf, abuf, sem):
        e0 = s * EPT + i * CH
        pltpu.make_async_copy(ei_hbm.at[pl.ds(e0, CH)], rv, rcs).wait()
        pltpu.make_async_copy(ei_hbm.at[pl.ds(N_EDGES + e0, CH)], cv, rcs).wait()
        for j in range(CH // 16):
            sl = pl.ds(j * 16, 16)
            r = rv[sl]
            cc = cv[sl]
            is_out = r < cc
            keep = (r >= base) & (r < base + HALF) & (r != cc)
            e = e0 + j * 16 + iota16
            gidx[sl] = jnp.where(keep, jnp.where(is_out, cc + N_NODES, cc), 0)
            aidx[sl] = jnp.where(keep, jnp.where(is_out, e + N_EDGES, e), 0)
            t = (r - base) * 2
            sidx[sl] = jnp.where(keep, jnp.where(is_out, t + 1, t),
                                 TRASH + s * 15 + j)
        pltpu.async_copy(p_hbm.at[gidx], pbuf, sem)
        pltpu.async_copy(a_hbm.at[aidx], abuf, sem)

    def _finish(gidx, aidx, sidx, pbuf, abuf, sem):
        pltpu.make_async_copy(p_hbm.at[gidx], pbuf, sem).wait()
        pltpu.make_async_copy(a_hbm.at[aidx], abuf, sem).wait()

        def _hrow(k, _):
            for j in range(8):
                sl = pl.ds(j * 16, 16)
                abuf[k, sl] = jnp.maximum(abuf[k, sl] + pbuf[k, sl], 0.0)
            return 0
        lax.fori_loop(0, CH, _hrow, 0)
        pltpu.sync_copy(abuf, acc.at[sidx], add=True)

    set0 = (gidx0, aidx0, sidx0, pbuf0, abuf0, sem0)
    set1 = (gidx1, aidx1, sidx1, pbuf1, abuf1, sem1)
    _rc(0, rv0, cv0, rcs0)
    _gissue(0, rv0, cv0, rcs0, *set0)
    _rc(1, rv1, cv1, rcs1)

    def _pair(t, _):
        k = t * 2
        _gissue(k + 1, rv1, cv1, rcs1, *set1)

        @pl.when(k + 2 < NCHUNK)
        def _():
            _rc(k + 2, rv0, cv0, rcs0)
        _finish(*set0)

        @pl.when(k + 2 < NCHUNK)
        def _():
            _gissue(k + 2, rv0, cv0, rcs0, *set0)

        @pl.when(k + 3 < NCHUNK)
        def _():
            _rc(k + 3, rv1, cv1, rcs1)
        _finish(*set1)
        return 0
    lax.fori_loop(0, NCHUNK // 2, _pair, 0)
    plsc.subcore_barrier()

    out_rows = ACC_ROWS // NS
    o0 = s * out_rows
    pltpu.sync_copy(acc.at[pl.ds(o0, out_rows)],
                    out_hbm.at[pl.ds(c * ACC_ROWS + o0, out_rows)])


@functools.lru_cache(maxsize=1)
def _sc_kernel():
    return functools.partial(
        pl.kernel,
        out_type=jax.ShapeDtypeStruct((2 * ACC_ROWS, D_OUT), jnp.float32),
        mesh=plsc.VectorSubcoreMesh(core_axis_name="c", subcore_axis_name="s",
                                    num_cores=NC, num_subcores=NS),
        scratch_types=[
            pltpu.VMEM((CH,), jnp.int32),
            pltpu.VMEM((CH,), jnp.int32),
            pltpu.VMEM((CH,), jnp.int32),
            pltpu.VMEM((CH,), jnp.int32),
            pltpu.VMEM((CH,), jnp.int32),
            pltpu.VMEM((CH,), jnp.int32),
            pltpu.VMEM((CH,), jnp.int32),
            pltpu.VMEM((CH,), jnp.int32),
            pltpu.VMEM((CH,), jnp.int32),
            pltpu.VMEM((CH,), jnp.int32),
            pltpu.VMEM((CH, D_OUT), jnp.float32),
            pltpu.VMEM((CH, D_OUT), jnp.float32),
            pltpu.VMEM((CH, D_OUT), jnp.float32),
            pltpu.VMEM((CH, D_OUT), jnp.float32),
            pltpu.VMEM((ZR, D_OUT), jnp.float32),
            pltpu.VMEM_SHARED((ACC_ROWS, D_OUT), jnp.float32),
            pltpu.SemaphoreType.DMA,
            pltpu.SemaphoreType.DMA,
            pltpu.SemaphoreType.DMA,
            pltpu.SemaphoreType.DMA,
        ],
    )(_sc_body)


EB = 6400
N_EB = N_EDGES // EB


@jax.jit
def _run(x, ei32, edge_attr, W_in, b_in, W_out, b_out):
    w_x = jnp.stack([W_in[:D_FEAT], W_out[:D_FEAT]])
    w_e = jnp.stack([W_in[D_FEAT:], W_out[D_FEAT:]])
    b = jnp.stack([b_in, b_out]).reshape(2, 1, D_OUT)

    p = pl.pallas_call(
        _p_body,
        grid=(2,),
        in_specs=[
            pl.BlockSpec((N_NODES, D_FEAT), lambda d: (0, 0)),
            pl.BlockSpec((1, D_FEAT, D_OUT), lambda d: (d, 0, 0)),
        ],
        out_specs=pl.BlockSpec((N_NODES, D_OUT), lambda d: (d, 0)),
        out_shape=jax.ShapeDtypeStruct((2 * N_NODES, D_OUT), jnp.float32),
    )(x, w_x)

    a = pl.pallas_call(
        _a_body,
        grid=(2, N_EB),
        in_specs=[
            pl.BlockSpec((D_EDGE, EB), lambda d, i: (0, i)),
            pl.BlockSpec((1, D_EDGE, D_OUT), lambda d, i: (d, 0, 0)),
            pl.BlockSpec((1, 1, D_OUT), lambda d, i: (d, 0, 0)),
        ],
        out_specs=pl.BlockSpec((EB, D_OUT), lambda d, i: (d * N_EB + i, 0)),
        out_shape=jax.ShapeDtypeStruct((2 * N_EDGES, D_OUT), jnp.float32),
    )(edge_attr.T, w_e, b)

    out = _sc_kernel()(p, a, ei32)
    lo = out[:N_NODES].reshape(HALF, 2 * D_OUT)
    hi = out[ACC_ROWS:ACC_ROWS + N_NODES].reshape(HALF, 2 * D_OUT)
    return jnp.concatenate([lo, hi], axis=0)


def kernel(unused, x, edge_index, edge_attr, W_in, b_in, W_out, b_out):
    ei32 = edge_index.astype(jnp.int32).reshape(2 * N_EDGES)
    return _run(x, ei32, edge_attr, W_in, b_in, W_out, b_out)


# speedup vs baseline: 21.5087x; 21.5087x over previous
import functools
import jax
import jax.numpy as jnp
from jax import lax
from jax.experimental import pallas as pl
from jax.experimental.pallas import tpu as pltpu
from jax.experimental.pallas import tpu_sc as plsc

N_NODES = 10000
N_EDGES = 320000
D_FEAT = 128
D_EDGE = 16
D_OUT = 128

NC = 2
NS = 16
HALF = N_NODES // NC
EPT = N_EDGES // NS
CH = 80
NCHUNK = EPT // CH
ACC_ROWS = 10240
TRASH = N_NODES
ZR = 32


def _p_body(x_ref, w_ref, p_ref):
    p_ref[...] = jnp.dot(x_ref[...], w_ref[0], preferred_element_type=jnp.float32)


def _a_body(eaT_ref, w_ref, b_ref, a_ref):
    a_ref[...] = (
        jax.lax.dot_general(eaT_ref[...], w_ref[0],
                            dimension_numbers=(((0,), (0,)), ((), ())),
                            preferred_element_type=jnp.float32)
        + b_ref[0]
    )


def _sc_body(p_hbm, a_hbm, ei_hbm, out_hbm,
             rv0, cv0, rv1, cv1,
             gidx0, aidx0, sidx0, gidx1, aidx1, sidx1,
             pbuf0, abuf0, pbuf1, abuf1,
             zbuf, acc, sem0, sem1, rcs0, rcs1):
    c = lax.axis_index("c")
    s = lax.axis_index("s")
    base = c * HALF
    iota16 = lax.iota(jnp.int32, 16)

    def _zrow(i, _):
        for j in range(8):
            zbuf[i, pl.ds(j * 16, 16)] = jnp.zeros((16,), jnp.float32)
        return 0
    lax.fori_loop(0, ZR, _zrow, 0)
    rows_per_tile = ACC_ROWS // NS
    z0 = s * rows_per_tile
    for t in range(rows_per_tile // ZR):
        pltpu.sync_copy(zbuf, acc.at[pl.ds(z0 + t * ZR, ZR)])
    plsc.subcore_barrier()

    def _rc(i, rv, cv, rcs):
        e0 = s * EPT + i * CH
        pltpu.async_copy(ei_hbm.at[pl.ds(e0, CH)], rv, rcs)
        pltpu.async_copy(ei_hbm.at[pl.ds(N_EDGES + e0, CH)], cv, rcs)

    def _gissue(i, rv, cv, rcs, gidx, aidx, sidx, pbuf, abuf, sem):
        e0 = s * EPT + i * CH
        pltpu.make_async_copy(ei_hbm.at[pl.ds(e0, CH)], rv, rcs).wait()
        pltpu.make_async_copy(ei_hbm.at[pl.ds(N_EDGES + e0, CH)], cv, rcs).wait()
        for j in range(CH // 16):
            sl = pl.ds(j * 16, 16)
            r = rv[sl]
            cc = cv[sl]
            is_out = r < cc
            keep = (r >= base) & (r < base + HALF) & (r != cc)
            e = e0 + j * 16 + iota16
            gidx[sl] = jnp.where(is_out, cc + N_NODES, cc)
            aidx[sl] = jnp.where(is_out, e + N_EDGES, e)
            t = (r - base) * 2
            sidx[sl] = jnp.where(keep, jnp.where(is_out, t + 1, t),
                                 TRASH + s * 15 + j)
        pltpu.async_copy(p_hbm.at[gidx], pbuf, sem)
        pltpu.async_copy(a_hbm.at[aidx], abuf, sem)

    def _finish(gidx, aidx, sidx, pbuf, abuf, sem):
        pltpu.make_async_copy(p_hbm.at[gidx], pbuf, sem).wait()
        pltpu.make_async_copy(a_hbm.at[aidx], abuf, sem).wait()

        def _hrow(k, _):
            for j in range(8):
                sl = pl.ds(j * 16, 16)
                abuf[k, sl] = jnp.maximum(abuf[k, sl] + pbuf[k, sl], 0.0)
            return 0
        lax.fori_loop(0, CH, _hrow, 0)
        pltpu.sync_copy(abuf, acc.at[sidx], add=True)

    set0 = (gidx0, aidx0, sidx0, pbuf0, abuf0, sem0)
    set1 = (gidx1, aidx1, sidx1, pbuf1, abuf1, sem1)
    _rc(0, rv0, cv0, rcs0)
    _gissue(0, rv0, cv0, rcs0, *set0)
    _rc(1, rv1, cv1, rcs1)

    def _pair(t, _):
        k = t * 2
        _gissue(k + 1, rv1, cv1, rcs1, *set1)

        @pl.when(k + 2 < NCHUNK)
        def _():
            _rc(k + 2, rv0, cv0, rcs0)
        _finish(*set0)

        @pl.when(k + 2 < NCHUNK)
        def _():
            _gissue(k + 2, rv0, cv0, rcs0, *set0)

        @pl.when(k + 3 < NCHUNK)
        def _():
            _rc(k + 3, rv1, cv1, rcs1)
        _finish(*set1)
        return 0
    lax.fori_loop(0, NCHUNK // 2, _pair, 0)
    plsc.subcore_barrier()

    out_rows = ACC_ROWS // NS
    o0 = s * out_rows
    pltpu.sync_copy(acc.at[pl.ds(o0, out_rows)],
                    out_hbm.at[pl.ds(c * ACC_ROWS + o0, out_rows)])


@functools.lru_cache(maxsize=1)
def _sc_kernel():
    return functools.partial(
        pl.kernel,
        out_type=jax.ShapeDtypeStruct((2 * ACC_ROWS, D_OUT), jnp.float32),
        mesh=plsc.VectorSubcoreMesh(core_axis_name="c", subcore_axis_name="s",
                                    num_cores=NC, num_subcores=NS),
        scratch_types=[
            pltpu.VMEM((CH,), jnp.int32),
            pltpu.VMEM((CH,), jnp.int32),
            pltpu.VMEM((CH,), jnp.int32),
            pltpu.VMEM((CH,), jnp.int32),
            pltpu.VMEM((CH,), jnp.int32),
            pltpu.VMEM((CH,), jnp.int32),
            pltpu.VMEM((CH,), jnp.int32),
            pltpu.VMEM((CH,), jnp.int32),
            pltpu.VMEM((CH,), jnp.int32),
            pltpu.VMEM((CH,), jnp.int32),
            pltpu.VMEM((CH, D_OUT), jnp.float32),
            pltpu.VMEM((CH, D_OUT), jnp.float32),
            pltpu.VMEM((CH, D_OUT), jnp.float32),
            pltpu.VMEM((CH, D_OUT), jnp.float32),
            pltpu.VMEM((ZR, D_OUT), jnp.float32),
            pltpu.VMEM_SHARED((ACC_ROWS, D_OUT), jnp.float32),
            pltpu.SemaphoreType.DMA,
            pltpu.SemaphoreType.DMA,
            pltpu.SemaphoreType.DMA,
            pltpu.SemaphoreType.DMA,
        ],
    )(_sc_body)


EB = 6400
N_EB = N_EDGES // EB


@jax.jit
def _run(x, ei32, edge_attr, W_in, b_in, W_out, b_out):
    w_x = jnp.stack([W_in[:D_FEAT], W_out[:D_FEAT]])
    w_e = jnp.stack([W_in[D_FEAT:], W_out[D_FEAT:]])
    b = jnp.stack([b_in, b_out]).reshape(2, 1, D_OUT)

    p = pl.pallas_call(
        _p_body,
        grid=(2,),
        in_specs=[
            pl.BlockSpec((N_NODES, D_FEAT), lambda d: (0, 0)),
            pl.BlockSpec((1, D_FEAT, D_OUT), lambda d: (d, 0, 0)),
        ],
        out_specs=pl.BlockSpec((N_NODES, D_OUT), lambda d: (d, 0)),
        out_shape=jax.ShapeDtypeStruct((2 * N_NODES, D_OUT), jnp.float32),
    )(x, w_x)

    a = pl.pallas_call(
        _a_body,
        grid=(2, N_EB),
        in_specs=[
            pl.BlockSpec((D_EDGE, EB), lambda d, i: (0, i)),
            pl.BlockSpec((1, D_EDGE, D_OUT), lambda d, i: (d, 0, 0)),
            pl.BlockSpec((1, 1, D_OUT), lambda d, i: (d, 0, 0)),
        ],
        out_specs=pl.BlockSpec((EB, D_OUT), lambda d, i: (d * N_EB + i, 0)),
        out_shape=jax.ShapeDtypeStruct((2 * N_EDGES, D_OUT), jnp.float32),
    )(edge_attr.T, w_e, b)

    out = _sc_kernel()(p, a, ei32)
    lo = out[:N_NODES].reshape(HALF, 2 * D_OUT)
    hi = out[ACC_ROWS:ACC_ROWS + N_NODES].reshape(HALF, 2 * D_OUT)
    return jnp.concatenate([lo, hi], axis=0)


def kernel(unused, x, edge_index, edge_attr, W_in, b_in, W_out, b_out):
    ei32 = edge_index.astype(jnp.int32).reshape(2 * N_EDGES)
    return _run(x, ei32, edge_attr, W_in, b_in, W_out, b_out)


# speedup vs baseline: 23.2875x; 1.0827x over previous
import functools
import jax
import jax.numpy as jnp
from jax import lax
from jax.experimental import pallas as pl
from jax.experimental.pallas import tpu as pltpu
from jax.experimental.pallas import tpu_sc as plsc

N_NODES = 10000
N_EDGES = 320000
D_FEAT = 128
D_EDGE = 16
D_OUT = 128

NC = 2
NS = 16
HALF = N_NODES // NC
EPT = N_EDGES // NS
CH = 80
NCHUNK = EPT // CH
ACC_ROWS = 10240
TRASH = N_NODES
ZR = 32


def _p_body(x_ref, w_ref, p_ref):
    p_ref[...] = jnp.dot(x_ref[...], w_ref[0], preferred_element_type=jnp.float32)


def _a_body(eaT_ref, w_ref, b_ref, a_ref):
    a_ref[...] = (
        jax.lax.dot_general(eaT_ref[...], w_ref[0],
                            dimension_numbers=(((0,), (0,)), ((), ())),
                            preferred_element_type=jnp.float32)
        + b_ref[0]
    )


def _sc_body(p_hbm, a_hbm, ei_hbm, out_hbm,
             rv0, cv0, rv1, cv1,
             gidx0, aidx0, sidx0, gidx1, aidx1, sidx1,
             pbuf0, abuf0, pbuf1, abuf1,
             zbuf, acc, sem0, sem1, rcs0, rcs1):
    c = lax.axis_index("c")
    s = lax.axis_index("s")
    base = c * HALF
    iota16 = lax.iota(jnp.int32, 16)

    def _zrow(i, _):
        for j in range(8):
            zbuf[i, pl.ds(j * 16, 16)] = jnp.zeros((16,), jnp.float32)
        return 0
    lax.fori_loop(0, ZR, _zrow, 0)
    rows_per_tile = ACC_ROWS // NS
    z0 = s * rows_per_tile
    for t in range(rows_per_tile // ZR):
        pltpu.sync_copy(zbuf, acc.at[pl.ds(z0 + t * ZR, ZR)])
    plsc.subcore_barrier()

    def _rc(i, rv, cv, rcs):
        e0 = s * EPT + i * CH
        pltpu.async_copy(ei_hbm.at[pl.ds(e0, CH)], rv, rcs)
        pltpu.async_copy(ei_hbm.at[pl.ds(N_EDGES + e0, CH)], cv, rcs)

    def _gissue(i, rv, cv, rcs, gidx, aidx, sidx, pbuf, abuf, sem):
        e0 = s * EPT + i * CH
        pltpu.make_async_copy(ei_hbm.at[pl.ds(e0, CH)], rv, rcs).wait()
        pltpu.make_async_copy(ei_hbm.at[pl.ds(N_EDGES + e0, CH)], cv, rcs).wait()
        for j in range(CH // 16):
            sl = pl.ds(j * 16, 16)
            r = rv[sl]
            cc = cv[sl]
            is_out = r < cc
            keep = (r >= base) & (r < base + HALF) & (r != cc)
            e = e0 + j * 16 + iota16
            gidx[sl] = jnp.where(is_out, cc + N_NODES, cc)
            aidx[sl] = jnp.where(is_out, e + N_EDGES, e)
            t = (r - base) * 2
            sidx[sl] = jnp.where(keep, jnp.where(is_out, t + 1, t),
                                 TRASH + s * 15 + j)
        pltpu.async_copy(p_hbm.at[gidx], pbuf, sem)
        pltpu.async_copy(a_hbm.at[aidx], abuf, sem)

    def _finish(gidx, aidx, sidx, pbuf, abuf, sem):
        pltpu.make_async_copy(p_hbm.at[gidx], pbuf, sem).wait()
        pltpu.make_async_copy(a_hbm.at[aidx], abuf, sem).wait()

        def _hrow(k, _):
            for j in range(8):
                sl = pl.ds(j * 16, 16)
                abuf[k, sl] = jnp.maximum(abuf[k, sl] + pbuf[k, sl], 0.0)
            return 0
        lax.fori_loop(0, CH, _hrow, 0)
        pltpu.sync_copy(abuf, acc.at[sidx], add=True)

    set0 = (gidx0, aidx0, sidx0, pbuf0, abuf0, sem0)
    set1 = (gidx1, aidx1, sidx1, pbuf1, abuf1, sem1)
    _rc(0, rv0, cv0, rcs0)
    _gissue(0, rv0, cv0, rcs0, *set0)
    _rc(1, rv1, cv1, rcs1)

    def _pair(t, _):
        k = t * 2
        _gissue(k + 1, rv1, cv1, rcs1, *set1)

        @pl.when(k + 2 < NCHUNK)
        def _():
            _rc(k + 2, rv0, cv0, rcs0)
        _finish(*set0)

        @pl.when(k + 2 < NCHUNK)
        def _():
            _gissue(k + 2, rv0, cv0, rcs0, *set0)

        @pl.when(k + 3 < NCHUNK)
        def _():
            _rc(k + 3, rv1, cv1, rcs1)
        _finish(*set1)
        return 0
    lax.fori_loop(0, NCHUNK // 2, _pair, 0)
    plsc.subcore_barrier()

    out_rows = ACC_ROWS // NS
    o0 = s * out_rows

    @pl.when(s < NS - 1)
    def _():
        pltpu.sync_copy(acc.at[pl.ds(o0, out_rows)],
                        out_hbm.at[pl.ds(c * N_NODES + o0, out_rows)])

    @pl.when(s == NS - 1)
    def _():
        last = N_NODES - (NS - 1) * out_rows
        pltpu.sync_copy(acc.at[pl.ds(o0, last)],
                        out_hbm.at[pl.ds(c * N_NODES + o0, last)])


@functools.lru_cache(maxsize=1)
def _sc_kernel():
    return functools.partial(
        pl.kernel,
        out_type=jax.ShapeDtypeStruct((2 * N_NODES, D_OUT), jnp.float32),
        mesh=plsc.VectorSubcoreMesh(core_axis_name="c", subcore_axis_name="s",
                                    num_cores=NC, num_subcores=NS),
        scratch_types=[
            pltpu.VMEM((CH,), jnp.int32),
            pltpu.VMEM((CH,), jnp.int32),
            pltpu.VMEM((CH,), jnp.int32),
            pltpu.VMEM((CH,), jnp.int32),
            pltpu.VMEM((CH,), jnp.int32),
            pltpu.VMEM((CH,), jnp.int32),
            pltpu.VMEM((CH,), jnp.int32),
            pltpu.VMEM((CH,), jnp.int32),
            pltpu.VMEM((CH,), jnp.int32),
            pltpu.VMEM((CH,), jnp.int32),
            pltpu.VMEM((CH, D_OUT), jnp.float32),
            pltpu.VMEM((CH, D_OUT), jnp.float32),
            pltpu.VMEM((CH, D_OUT), jnp.float32),
            pltpu.VMEM((CH, D_OUT), jnp.float32),
            pltpu.VMEM((ZR, D_OUT), jnp.float32),
            pltpu.VMEM_SHARED((ACC_ROWS, D_OUT), jnp.float32),
            pltpu.SemaphoreType.DMA,
            pltpu.SemaphoreType.DMA,
            pltpu.SemaphoreType.DMA,
            pltpu.SemaphoreType.DMA,
        ],
    )(_sc_body)


EB = 12800
N_EB = N_EDGES // EB


@jax.jit
def _run(x, ei32, edge_attr, W_in, b_in, W_out, b_out):
    w_x = jnp.stack([W_in[:D_FEAT], W_out[:D_FEAT]])
    w_e = jnp.stack([W_in[D_FEAT:], W_out[D_FEAT:]])
    b = jnp.stack([b_in, b_out]).reshape(2, 1, D_OUT)

    p = pl.pallas_call(
        _p_body,
        grid=(2,),
        in_specs=[
            pl.BlockSpec((N_NODES, D_FEAT), lambda d: (0, 0)),
            pl.BlockSpec((1, D_FEAT, D_OUT), lambda d: (d, 0, 0)),
        ],
        out_specs=pl.BlockSpec((N_NODES, D_OUT), lambda d: (d, 0)),
        out_shape=jax.ShapeDtypeStruct((2 * N_NODES, D_OUT), jnp.float32),
    )(x, w_x)

    a = pl.pallas_call(
        _a_body,
        grid=(2, N_EB),
        in_specs=[
            pl.BlockSpec((D_EDGE, EB), lambda d, i: (0, i)),
            pl.BlockSpec((1, D_EDGE, D_OUT), lambda d, i: (d, 0, 0)),
            pl.BlockSpec((1, 1, D_OUT), lambda d, i: (d, 0, 0)),
        ],
        out_specs=pl.BlockSpec((EB, D_OUT), lambda d, i: (d * N_EB + i, 0)),
        out_shape=jax.ShapeDtypeStruct((2 * N_EDGES, D_OUT), jnp.float32),
    )(edge_attr.T, w_e, b)

    out = _sc_kernel()(p, a, ei32)
    return out.reshape(N_NODES, 2 * D_OUT)


def kernel(unused, x, edge_index, edge_attr, W_in, b_in, W_out, b_out):
    ei32 = edge_index.astype(jnp.int32).reshape(2 * N_EDGES)
    return _run(x, ei32, edge_attr, W_in, b_in, W_out, b_out)


# speedup vs baseline: 23.2980x; 1.0005x over previous
import functools
import jax
import jax.numpy as jnp
import numpy as np
from jax import lax
from jax.experimental import pallas as pl
from jax.experimental.pallas import tpu as pltpu
from jax.experimental.pallas import tpu_sc as plsc

N_NODES = 10000
N_EDGES = 320000
D_FEAT = 128
D_EDGE = 16
D_OUT = 128

NC = 2
NS = 16
HALF = N_NODES // NC
EPT = N_EDGES // NS
CH = 80
NCHUNK = EPT // CH
ACC_ROWS = 10240
TRASH = N_NODES
ZR = 32


def _p_body(x_ref, w_ref, p_ref):
    p_ref[...] = jnp.dot(x_ref[...], w_ref[0], preferred_element_type=jnp.float32)


def _a_body(eaT_ref, w_ref, b_ref, a_ref):
    a_ref[...] = (
        jax.lax.dot_general(eaT_ref[...], w_ref[0],
                            dimension_numbers=(((0,), (0,)), ((), ())),
                            preferred_element_type=jnp.float32)
        + b_ref[0]
    )


def _sc_body(p_hbm, a_hbm, ei_hbm, out_hbm,
             rv0, cv0, rv1, cv1,
             gidx0, aidx0, sidx0, gidx1, aidx1, sidx1,
             pbuf0, abuf0, pbuf1, abuf1,
             zbuf, acc, sem0, sem1, rcs0, rcs1):
    c = lax.axis_index("c")
    s = lax.axis_index("s")
    base = c * HALF
    iota16 = lax.iota(jnp.int32, 16)

    def _zrow(i, _):
        for j in range(8):
            zbuf[i, pl.ds(j * 16, 16)] = jnp.zeros((16,), jnp.float32)
        return 0
    lax.fori_loop(0, ZR, _zrow, 0)
    rows_per_tile = ACC_ROWS // NS
    z0 = s * rows_per_tile
    for t in range(rows_per_tile // ZR):
        pltpu.sync_copy(zbuf, acc.at[pl.ds(z0 + t * ZR, ZR)])
    plsc.subcore_barrier()

    def _rc(i, rv, cv, rcs):
        e0 = s * EPT + i * CH
        pltpu.async_copy(ei_hbm.at[pl.ds(e0, CH)], rv, rcs)
        pltpu.async_copy(ei_hbm.at[pl.ds(N_EDGES + e0, CH)], cv, rcs)

    def _gissue(i, rv, cv, rcs, gidx, aidx, sidx, pbuf, abuf, sem):
        e0 = s * EPT + i * CH
        pltpu.make_async_copy(ei_hbm.at[pl.ds(e0, CH)], rv, rcs).wait()
        pltpu.make_async_copy(ei_hbm.at[pl.ds(N_EDGES + e0, CH)], cv, rcs).wait()
        for j in range(CH // 16):
            sl = pl.ds(j * 16, 16)
            r = rv[sl]
            cc = cv[sl]
            is_out = r < cc
            keep = (r >= base) & (r < base + HALF) & (r != cc)
            e = e0 + j * 16 + iota16
            gidx[sl] = jnp.where(is_out, cc + N_NODES, cc)
            aidx[sl] = jnp.where(is_out, e + N_EDGES, e)
            t = (r - base) * 2
            sidx[sl] = jnp.where(keep, jnp.where(is_out, t + 1, t),
                                 TRASH + s * 15 + j)
        pltpu.async_copy(p_hbm.at[gidx], pbuf, sem)
        pltpu.async_copy(a_hbm.at[aidx], abuf, sem)

    def _finish(gidx, aidx, sidx, pbuf, abuf, sem):
        pltpu.make_async_copy(p_hbm.at[gidx], pbuf, sem).wait()
        pltpu.make_async_copy(a_hbm.at[aidx], abuf, sem).wait()

        def _hrow(k, _):
            for j in range(8):
                sl = pl.ds(j * 16, 16)
                abuf[k, sl] = jnp.maximum(abuf[k, sl] + pbuf[k, sl], 0.0)
            return 0
        lax.fori_loop(0, CH, _hrow, 0)
        pltpu.sync_copy(abuf, acc.at[sidx], add=True)

    set0 = (gidx0, aidx0, sidx0, pbuf0, abuf0, sem0)
    set1 = (gidx1, aidx1, sidx1, pbuf1, abuf1, sem1)
    _rc(0, rv0, cv0, rcs0)
    _gissue(0, rv0, cv0, rcs0, *set0)
    _rc(1, rv1, cv1, rcs1)

    def _pair(t, _):
        k = t * 2
        _gissue(k + 1, rv1, cv1, rcs1, *set1)

        @pl.when(k + 2 < NCHUNK)
        def _():
            _rc(k + 2, rv0, cv0, rcs0)
        _finish(*set0)

        @pl.when(k + 2 < NCHUNK)
        def _():
            _gissue(k + 2, rv0, cv0, rcs0, *set0)

        @pl.when(k + 3 < NCHUNK)
        def _():
            _rc(k + 3, rv1, cv1, rcs1)
        _finish(*set1)
        return 0
    lax.fori_loop(0, NCHUNK // 2, _pair, 0)
    plsc.subcore_barrier()

    out_rows = ACC_ROWS // NS
    o0 = s * out_rows

    @pl.when(s < NS - 1)
    def _():
        pltpu.sync_copy(acc.at[pl.ds(o0, out_rows)],
                        out_hbm.at[pl.ds(c * N_NODES + o0, out_rows)])

    @pl.when(s == NS - 1)
    def _():
        last = N_NODES - (NS - 1) * out_rows
        pltpu.sync_copy(acc.at[pl.ds(o0, last)],
                        out_hbm.at[pl.ds(c * N_NODES + o0, last)])


@functools.lru_cache(maxsize=1)
def _sc_kernel():
    return functools.partial(
        pl.kernel,
        out_type=jax.ShapeDtypeStruct((2 * N_NODES, D_OUT), jnp.float32),
        mesh=plsc.VectorSubcoreMesh(core_axis_name="c", subcore_axis_name="s",
                                    num_cores=NC, num_subcores=NS),
        scratch_types=[
            pltpu.VMEM((CH,), jnp.int32),
            pltpu.VMEM((CH,), jnp.int32),
            pltpu.VMEM((CH,), jnp.int32),
            pltpu.VMEM((CH,), jnp.int32),
            pltpu.VMEM((CH,), jnp.int32),
            pltpu.VMEM((CH,), jnp.int32),
            pltpu.VMEM((CH,), jnp.int32),
            pltpu.VMEM((CH,), jnp.int32),
            pltpu.VMEM((CH,), jnp.int32),
            pltpu.VMEM((CH,), jnp.int32),
            pltpu.VMEM((CH, D_OUT), jnp.float32),
            pltpu.VMEM((CH, D_OUT), jnp.float32),
            pltpu.VMEM((CH, D_OUT), jnp.float32),
            pltpu.VMEM((CH, D_OUT), jnp.float32),
            pltpu.VMEM((ZR, D_OUT), jnp.float32),
            pltpu.VMEM_SHARED((ACC_ROWS, D_OUT), jnp.float32),
            pltpu.SemaphoreType.DMA,
            pltpu.SemaphoreType.DMA,
            pltpu.SemaphoreType.DMA,
            pltpu.SemaphoreType.DMA,
        ],
    )(_sc_body)


EB = 12800
N_EB = N_EDGES // EB


@jax.jit
def _run(x, ei32, edge_attr, W_in, b_in, W_out, b_out):
    w_x = jnp.stack([W_in[:D_FEAT], W_out[:D_FEAT]])
    w_e = jnp.stack([W_in[D_FEAT:], W_out[D_FEAT:]])
    b = jnp.stack([b_in, b_out]).reshape(2, 1, D_OUT)

    p = pl.pallas_call(
        _p_body,
        grid=(2,),
        in_specs=[
            pl.BlockSpec((N_NODES, D_FEAT), lambda d: (0, 0)),
            pl.BlockSpec((1, D_FEAT, D_OUT), lambda d: (d, 0, 0)),
        ],
        out_specs=pl.BlockSpec((N_NODES, D_OUT), lambda d: (d, 0)),
        out_shape=jax.ShapeDtypeStruct((2 * N_NODES, D_OUT), jnp.float32),
    )(x, w_x)

    a = pl.pallas_call(
        _a_body,
        grid=(2, N_EB),
        in_specs=[
            pl.BlockSpec((D_EDGE, EB), lambda d, i: (0, i)),
            pl.BlockSpec((1, D_EDGE, D_OUT), lambda d, i: (d, 0, 0)),
            pl.BlockSpec((1, 1, D_OUT), lambda d, i: (d, 0, 0)),
        ],
        out_specs=pl.BlockSpec((EB, D_OUT), lambda d, i: (d * N_EB + i, 0)),
        out_shape=jax.ShapeDtypeStruct((2 * N_EDGES, D_OUT), jnp.float32),
    )(edge_attr.T, w_e, b)

    out = _sc_kernel()(p, a, ei32)
    return out.reshape(N_NODES, 2 * D_OUT)


def kernel(unused, x, edge_index, edge_attr, W_in, b_in, W_out, b_out):
    ei32 = edge_index.astype(jnp.int32).reshape(2 * N_EDGES)
    return _run(x, ei32, edge_attr, W_in, b_in, W_out, b_out)
